# baseline (speedup 1.0000x reference)
import functools

import jax
import jax.numpy as jnp
from jax import lax
from jax.experimental import pallas as pl
from jax.experimental.pallas import tpu as pltpu
from jax.experimental.pallas import tpu_sc as plsc

NC = 2
NS = 16
NW = NC * NS
GRP = 16


def _sc_gather(user_idx, item_idx, u3, i3):
    B = user_idx.shape[0]
    D = 64
    W = u3.shape[2]
    bpw = B // NW
    ngrp = bpw // GRP
    uidx = user_idx.reshape(NW, bpw)
    iidx = item_idx.reshape(NW, bpw)
    mesh = plsc.VectorSubcoreMesh(core_axis_name="c", subcore_axis_name="s")

    @functools.partial(
        pl.kernel,
        out_type=(
            jax.ShapeDtypeStruct((NW, bpw, D), jnp.float32),
            jax.ShapeDtypeStruct((NW, bpw, D), jnp.float32),
        ),
        mesh=mesh,
        compiler_params=pltpu.CompilerParams(use_tc_tiling_on_sc=True,
                                             needs_layout_passes=False),
        scratch_types=[
            pltpu.VMEM((bpw,), jnp.int32),
            pltpu.VMEM((bpw,), jnp.int32),
            pltpu.VMEM((2, GRP * 8, W), jnp.float32),
            pltpu.VMEM((2, GRP * 8, W), jnp.float32),
            pltpu.VMEM((2, GRP, D), jnp.float32),
            pltpu.VMEM((2, GRP, D), jnp.float32),
            pltpu.SemaphoreType.DMA,
            pltpu.SemaphoreType.DMA,
            pltpu.SemaphoreType.DMA,
            pltpu.SemaphoreType.DMA,
        ],
    )
    def gather_k(uidx_hbm, iidx_hbm, utab_hbm, itab_hbm, uout_hbm, iout_hbm,
                 uidx_v, iidx_v, ublk_v, iblk_v, uext_v, iext_v,
                 usem, isem, uwsem, iwsem):
        wid = lax.axis_index("s") * NC + lax.axis_index("c")
        pltpu.sync_copy(uidx_hbm.at[wid], uidx_v)
        pltpu.sync_copy(iidx_hbm.at[wid], iidx_v)
        lanes = lax.iota(jnp.int32, 16)

        def w_drain(ext_v, out_hbm, wsem):
            pltpu.make_async_copy(
                ext_v.at[0],
                out_hbm.at[wid, pl.ds(0, GRP), :], wsem).wait()

        def body(g, _):
            uvec = uidx_v[pl.ds(g * GRP, GRP)]
            ivec = iidx_v[pl.ds(g * GRP, GRP)]

            @pl.when(g >= 2)
            def _():
                w_drain(uext_v, uout_hbm, uwsem)
                w_drain(iext_v, iout_hbm, iwsem)

            work = []
            for vec, tab_hbm, blk_v, sem in ((uvec, utab_hbm, ublk_v, usem),
                                             (ivec, itab_hbm, iblk_v, isem)):
                cps = []
                rs = []
                for l in range(GRP):
                    r = lax.reduce_max(jnp.where(lanes == l, vec, 0), axes=(0,))
                    rs.append(r)
                    cps.append(pltpu.async_copy(
                        tab_hbm.at[r >> 4],
                        blk_v.at[g % 2, pl.ds(pl.multiple_of(l * 8, 8), 8), :],
                        sem))
                work.append((cps, rs))

            for (cps, rs), blk_v, ext_v, out_hbm, wsem in (
                    (work[0], ublk_v, uext_v, uout_hbm, uwsem),
                    (work[1], iblk_v, iext_v, iout_hbm, iwsem)):
                for cp in cps:
                    cp.wait()
                blk = blk_v.at[g % 2]
                for l in range(GRP):
                    r = rs[l]
                    row = l * 8 + ((r >> 1) & 7)
                    cbase = (r & 1) * D
                    rowv = jnp.broadcast_to(row, (16,))
                    for k in range(D // 16):
                        vals = plsc.load_gather(
                            blk, [rowv, cbase + lanes + k * 16])
                        ext_v[g % 2, l, pl.ds(k * 16, 16)] = vals
                pltpu.async_copy(
                    ext_v.at[g % 2],
                    out_hbm.at[wid, pl.ds(g * GRP, GRP), :], wsem)
            return 0

        lax.fori_loop(0, ngrp, body, 0)
        for _ in range(2):
            w_drain(uext_v, uout_hbm, uwsem)
            w_drain(iext_v, iout_hbm, iwsem)

    u_rows, i_rows = gather_k(uidx, iidx, u3, i3)
    return u_rows.reshape(B, D), i_rows.reshape(B, D)


def _mlp_body(u_ref, i_ref, f_ref, ltab_ref, w1a_ref, w1b_ref, w1c_ref,
              b1_ref, w2_ref, b2_ref, uo_ref, io_ref):
    u = u_ref[...]
    n = jnp.sqrt(jnp.sum(u * u, axis=1, keepdims=True))
    uo_ref[...] = u / jnp.maximum(n, 1e-12)

    f = f_ref[...]
    lidx = jnp.clip(f[:, 2:3], 0.0, None).astype(jnp.int32)
    classes = lax.broadcasted_iota(jnp.int32, (1, ltab_ref.shape[0]), 1)
    onehot = (lidx == classes).astype(jnp.float32)
    lang = jnp.dot(onehot, ltab_ref[...],
                   preferred_element_type=jnp.float32)
    x = (jnp.dot(i_ref[...], w1a_ref[...], preferred_element_type=jnp.float32)
         + jnp.dot(lang, w1b_ref[...], preferred_element_type=jnp.float32)
         + f[:, 0:1] * w1c_ref[0:1, :] + f[:, 1:2] * w1c_ref[1:2, :]
         + b1_ref[...])
    h = jnp.maximum(x, 0.0)
    o = jnp.dot(h, w2_ref[...], preferred_element_type=jnp.float32) + b2_ref[...]
    n2 = jnp.sqrt(jnp.sum(o * o, axis=1, keepdims=True))
    io_ref[...] = o / jnp.maximum(n2, 1e-12)


def _tc_mlp(u_rows, i_rows, item_feats, lang_table, W1, b1, W2, b2):
    B, D = u_rows.shape
    L = lang_table.shape[0]
    E = lang_table.shape[1]
    BB = 2048
    grid = (B // BB,)
    w1a = W1[:, :D].T
    w1b = W1[:, D:D + E].T
    w1c = W1[:, D + E:].T
    b1r = b1.reshape(1, D)
    w2t = W2.T
    b2r = b2.reshape(1, D)
    full = lambda shape: pl.BlockSpec(shape, lambda b: (0, 0))
    return pl.pallas_call(
        _mlp_body,
        grid=grid,
        in_specs=[
            pl.BlockSpec((BB, D), lambda b: (b, 0)),
            pl.BlockSpec((BB, D), lambda b: (b, 0)),
            pl.BlockSpec((BB, 3), lambda b: (b, 0)),
            full((L, E)),
            full((D, D)),
            full((E, D)),
            full((2, D)),
            full((1, D)),
            full((D, D)),
            full((1, D)),
        ],
        out_specs=[
            pl.BlockSpec((BB, D), lambda b: (b, 0)),
            pl.BlockSpec((BB, D), lambda b: (b, 0)),
        ],
        out_shape=[
            jax.ShapeDtypeStruct((B, D), jnp.float32),
            jax.ShapeDtypeStruct((B, D), jnp.float32),
        ],
    )(u_rows, i_rows, item_feats, lang_table, w1a, w1b, w1c, b1r, w2t, b2r)


def kernel(user_idx, item_idx, item_feats, user_table, item_table, lang_table,
           W1, b1, W2, b2):
    V, D = user_table.shape
    u3 = user_table.reshape(V // 16, 8, 2 * D)
    i3 = item_table.reshape(V // 16, 8, 2 * D)
    u_rows, i_rows = _sc_gather(user_idx, item_idx, u3, i3)
    u, i = _tc_mlp(u_rows, i_rows, item_feats, lang_table, W1, b1, W2, b2)
    return (u, i)


# speedup vs baseline: 1.9166x; 1.9166x over previous
import functools

import jax
import jax.numpy as jnp
from jax import lax
from jax.experimental import pallas as pl
from jax.experimental.pallas import tpu as pltpu
from jax.experimental.pallas import tpu_sc as plsc

NC = 2
NS = 16
NW = NC * NS
GRP = 16
DEPTH = 4


def _sc_gather(user_idx, item_idx, u3, item_table):
    B = user_idx.shape[0]
    D = item_table.shape[1]
    bpw = B // NW
    ngrp = bpw // GRP
    uidx = user_idx.reshape(NW, bpw)
    iidx = item_idx.reshape(NW, bpw)
    mesh = plsc.VectorSubcoreMesh(core_axis_name="c", subcore_axis_name="s")

    @functools.partial(
        pl.kernel,
        out_type=(
            jax.ShapeDtypeStruct((NW, bpw * 8, D), jnp.float32),
            jax.ShapeDtypeStruct((NW, bpw, D), jnp.float32),
        ),
        mesh=mesh,
        compiler_params=pltpu.CompilerParams(use_tc_tiling_on_sc=True,
                                             needs_layout_passes=False),
        scratch_types=[
            pltpu.VMEM((bpw,), jnp.int32),
            pltpu.VMEM((bpw,), jnp.int32),
            pltpu.VMEM((bpw, D), jnp.float32),
            pltpu.VMEM((2, GRP * 8, D), jnp.float32),
            pltpu.SemaphoreType.DMA,
            pltpu.SemaphoreType.DMA,
            pltpu.SemaphoreType.DMA,
        ],
    )
    def gather_k(uidx_hbm, iidx_hbm, utab_hbm, itab_hbm, uout_hbm, iout_hbm,
                 uidx_v, iidx_v, irows_v, ublk_v, isem, usem, wsem):
        wid = lax.axis_index("s") * NC + lax.axis_index("c")
        pltpu.sync_copy(uidx_hbm.at[wid], uidx_v)
        pltpu.sync_copy(iidx_hbm.at[wid], iidx_v)
        lanes = lax.iota(jnp.int32, 16)

        def i_drain():
            pltpu.make_async_copy(itab_hbm.at[pl.ds(0, 1), :],
                                  irows_v.at[pl.ds(0, 1), :], isem).wait()

        def w_drain():
            pltpu.make_async_copy(
                ublk_v.at[0],
                uout_hbm.at[wid, pl.ds(0, GRP * 8), :], wsem).wait()

        def body(g, _):
            ivec = iidx_v[pl.ds(g * GRP, GRP)]
            uvec = uidx_v[pl.ds(g * GRP, GRP)]
            for l in range(GRP):
                r = lax.reduce_max(jnp.where(lanes == l, ivec, 0), axes=(0,))
                pltpu.async_copy(
                    itab_hbm.at[pl.ds(r, 1), :],
                    irows_v.at[pl.ds(g * GRP + l, 1), :], isem)

            @pl.when(g >= DEPTH)
            def _():
                for _l in range(GRP):
                    i_drain()

            @pl.when(g >= 2)
            def _():
                w_drain()

            ucps = []
            for l in range(GRP):
                q = lax.reduce_max(jnp.where(lanes == l, uvec, 0),
                                   axes=(0,)) >> 3
                ucps.append(pltpu.async_copy(
                    utab_hbm.at[q],
                    ublk_v.at[g % 2, pl.ds(pl.multiple_of(l * 8, 8), 8), :],
                    usem))
            for cp in ucps:
                cp.wait()
            pltpu.async_copy(
                ublk_v.at[g % 2],
                uout_hbm.at[wid,
                            pl.ds(pl.multiple_of(g * GRP * 8, 8), GRP * 8), :],
                wsem)
            return 0

        lax.fori_loop(0, ngrp, body, 0)
        for _ in range(DEPTH * GRP):
            i_drain()
        for _ in range(2):
            w_drain()
        pltpu.sync_copy(irows_v, iout_hbm.at[wid])

    u_blk, i_rows = gather_k(uidx, iidx, u3, item_table)
    return u_blk.reshape(B, 8, D), i_rows.reshape(B, D)


def _mlp_body(u8_ref, i_ref, up_ref, f_ref, ltab_ref,
              w1a_ref, w1b_ref, w1c_ref, b1_ref, w2_ref, b2_ref,
              uo_ref, io_ref):
    D = uo_ref.shape[1]
    BB = uo_ref.shape[0]
    u8 = u8_ref[...]
    up = up_ref[...]
    u = jnp.zeros((BB, D), jnp.float32)
    for s in range(8):
        u = u + jnp.where(up == s, u8[:, s, :], 0.0)
    n = jnp.sqrt(jnp.sum(u * u, axis=1, keepdims=True))
    uo_ref[...] = u / jnp.maximum(n, 1e-12)

    f = f_ref[...]
    lidx = jnp.clip(f[:, 2:3], 0.0, None).astype(jnp.int32)
    classes = lax.broadcasted_iota(jnp.int32, (1, ltab_ref.shape[0]), 1)
    onehot = (lidx == classes).astype(jnp.float32)
    lang = jnp.dot(onehot, ltab_ref[...],
                   preferred_element_type=jnp.float32)
    x = (jnp.dot(i_ref[...], w1a_ref[...], preferred_element_type=jnp.float32)
         + jnp.dot(lang, w1b_ref[...], preferred_element_type=jnp.float32)
         + f[:, 0:1] * w1c_ref[0:1, :] + f[:, 1:2] * w1c_ref[1:2, :]
         + b1_ref[...])
    h = jnp.maximum(x, 0.0)
    o = jnp.dot(h, w2_ref[...], preferred_element_type=jnp.float32) + b2_ref[...]
    n2 = jnp.sqrt(jnp.sum(o * o, axis=1, keepdims=True))
    io_ref[...] = o / jnp.maximum(n2, 1e-12)


def _tc_mlp(u_blk, i_rows, u_par, item_feats, lang_table, W1, b1, W2, b2):
    B = u_blk.shape[0]
    D = u_blk.shape[2]
    L = lang_table.shape[0]
    E = lang_table.shape[1]
    BB = 2048
    grid = (B // BB,)
    w1a = W1[:, :D].T
    w1b = W1[:, D:D + E].T
    w1c = W1[:, D + E:].T
    b1r = b1.reshape(1, D)
    w2t = W2.T
    b2r = b2.reshape(1, D)
    full = lambda shape: pl.BlockSpec(shape, lambda b: tuple(0 for _ in shape))
    return pl.pallas_call(
        _mlp_body,
        grid=grid,
        in_specs=[
            pl.BlockSpec((BB, 8, D), lambda b: (b, 0, 0)),
            pl.BlockSpec((BB, D), lambda b: (b, 0)),
            pl.BlockSpec((BB, 1), lambda b: (b, 0)),
            pl.BlockSpec((BB, 3), lambda b: (b, 0)),
            full((L, E)),
            full((D, D)),
            full((E, D)),
            full((2, D)),
            full((1, D)),
            full((D, D)),
            full((1, D)),
        ],
        out_specs=[
            pl.BlockSpec((BB, D), lambda b: (b, 0)),
            pl.BlockSpec((BB, D), lambda b: (b, 0)),
        ],
        out_shape=[
            jax.ShapeDtypeStruct((B, D), jnp.float32),
            jax.ShapeDtypeStruct((B, D), jnp.float32),
        ],
    )(u_blk, i_rows, u_par, item_feats, lang_table,
      w1a, w1b, w1c, b1r, w2t, b2r)


def kernel(user_idx, item_idx, item_feats, user_table, item_table, lang_table,
           W1, b1, W2, b2):
    V, D = user_table.shape
    u3 = user_table.reshape(V // 8, 8, D)
    u_blk, i_rows = _sc_gather(user_idx, item_idx, u3, item_table)
    u_par = (user_idx & 7).reshape(-1, 1)
    u, i = _tc_mlp(u_blk, i_rows, u_par, item_feats, lang_table,
                   W1, b1, W2, b2)
    return (u, i)
